# baseline (speedup 1.0000x reference)
import functools

import jax
import jax.numpy as jnp
from jax import lax
from jax.experimental import pallas as pl
from jax.experimental.pallas import tpu as pltpu
from jax.experimental.pallas import tpu_sc as plsc

_EMB_DIM = 64
_FEAT_DIM = 128
_HID = 64
_BATCH = 16384

_NC = 2
_NS = 16
_NW = _NC * _NS
_B_PER_W = _BATCH // _NW
_CHUNK = 128
_NCHUNK = _B_PER_W // _CHUNK


@functools.cache
def _make_sc_gather():
    @functools.partial(
        pl.kernel,
        mesh=plsc.VectorSubcoreMesh(core_axis_name="c", subcore_axis_name="s"),
        compiler_params=pltpu.CompilerParams(use_tc_tiling_on_sc=False),
        out_type=jax.ShapeDtypeStruct((_BATCH, _EMB_DIM), jnp.float32),
        scratch_types=[
            pltpu.VMEM((_NCHUNK, _CHUNK), jnp.int32),
            pltpu.VMEM((_B_PER_W, _EMB_DIM), jnp.float32),
            pltpu.SemaphoreType.DMA,
        ],
    )
    def _sc_gather(table_hbm, idx_hbm, out_hbm, idx_v, rows_v, sem):
        wid = lax.axis_index("s") * _NC + lax.axis_index("c")
        base = wid * _B_PER_W
        pltpu.sync_copy(idx_hbm.at[wid], idx_v)
        copies = []
        for j in range(_NCHUNK):
            copies.append(
                pltpu.async_copy(
                    table_hbm.at[idx_v.at[j]],
                    rows_v.at[pl.ds(j * _CHUNK, _CHUNK)],
                    sem,
                )
            )
        for c in copies:
            c.wait()
        pltpu.sync_copy(rows_v, out_hbm.at[pl.ds(base, _B_PER_W)])

    return _sc_gather


_BB = 1024


def _tc_body(feat_ref, w_ref, b_ref, emb_ref, out_ref):
    proj = lax.dot_general(
        feat_ref[...], w_ref[...],
        (((1,), (1,)), ((), ())),
        preferred_element_type=jnp.float32,
    )
    out_ref[:, :_HID] = proj + b_ref[...]
    out_ref[:, _HID:] = emb_ref[...]


def kernel(feat, id, W, b, table):
    idx = id.astype(jnp.int32).reshape(_NW, _NCHUNK, _CHUNK)
    emb = _make_sc_gather()(table, idx)
    out = pl.pallas_call(
        _tc_body,
        grid=(_BATCH // _BB,),
        in_specs=[
            pl.BlockSpec((_BB, _FEAT_DIM), lambda i: (i, 0)),
            pl.BlockSpec((_HID, _FEAT_DIM), lambda i: (0, 0)),
            pl.BlockSpec((1, _HID), lambda i: (0, 0)),
            pl.BlockSpec((_BB, _EMB_DIM), lambda i: (i, 0)),
        ],
        out_specs=pl.BlockSpec((_BB, _HID + _EMB_DIM), lambda i: (i, 0)),
        out_shape=jax.ShapeDtypeStruct((_BATCH, _HID + _EMB_DIM), jnp.float32),
    )(feat, W, b.reshape(1, _HID), emb)
    return out


# speedup vs baseline: 1.7004x; 1.7004x over previous
import functools

import jax
import jax.numpy as jnp
from jax import lax
from jax.experimental import pallas as pl
from jax.experimental.pallas import tpu as pltpu
from jax.experimental.pallas import tpu_sc as plsc

_VOCAB = 1000000
_EMB_DIM = 64
_FEAT_DIM = 128
_HID = 64
_BATCH = 16384

_NC = 2
_NS = 16
_NW = _NC * _NS
_B_PER_W = _BATCH // _NW
_CHUNK = 64
_NCHUNK = _B_PER_W // _CHUNK
_NBLK = _VOCAB // 8


@functools.cache
def _make_sc_gather():
    @functools.partial(
        pl.kernel,
        mesh=plsc.VectorSubcoreMesh(core_axis_name="c", subcore_axis_name="s"),
        out_type=jax.ShapeDtypeStruct((_BATCH, _EMB_DIM), jnp.float32),
        scratch_types=[
            pltpu.VMEM((_B_PER_W,), jnp.int32),
            pltpu.VMEM((_B_PER_W, _EMB_DIM), jnp.float32),
            pltpu.SemaphoreType.DMA,
        ],
    )
    def _sc_gather(table_hbm, idx_hbm, out_hbm, idx_v, rows_v, sem):
        wid = lax.axis_index("s") * _NC + lax.axis_index("c")
        base = wid * _B_PER_W
        pltpu.sync_copy(idx_hbm.at[wid], idx_v)

        def _fire(g, carry):
            vec = idx_v[pl.ds(g * 16, 16)]
            for u in range(16):
                pltpu.async_copy(
                    table_hbm.at[pl.ds(vec[u], 1)],
                    rows_v.at[pl.ds(g * 16 + u, 1)],
                    sem,
                )
            return carry

        lax.fori_loop(0, _B_PER_W // 16, _fire, 0)

        def _drain(i, carry):
            pltpu.make_async_copy(
                table_hbm.at[pl.ds(0, 1)], rows_v.at[pl.ds(i, 1)], sem
            ).wait()
            return carry

        lax.fori_loop(0, _B_PER_W, _drain, 0, unroll=8)
        pltpu.sync_copy(rows_v, out_hbm.at[pl.ds(base, _B_PER_W)])

    return _sc_gather


_BB = 1024


def _tc_body(feat_ref, w_ref, b_ref, emb_ref, out_ref):
    proj = lax.dot_general(
        feat_ref[...], w_ref[...],
        (((1,), (1,)), ((), ())),
        preferred_element_type=jnp.float32,
    )
    out_ref[:, :_HID] = proj + b_ref[...]
    out_ref[:, _HID:] = emb_ref[...]


def kernel(feat, id, W, b, table):
    ids = id.astype(jnp.int32).reshape(_NW, _B_PER_W)
    emb = _make_sc_gather()(table, ids)
    out = pl.pallas_call(
        _tc_body,
        grid=(_BATCH // _BB,),
        in_specs=[
            pl.BlockSpec((_BB, _FEAT_DIM), lambda i: (i, 0)),
            pl.BlockSpec((_HID, _FEAT_DIM), lambda i: (0, 0)),
            pl.BlockSpec((1, _HID), lambda i: (0, 0)),
            pl.BlockSpec((_BB, _EMB_DIM), lambda i: (i, 0)),
        ],
        out_specs=pl.BlockSpec((_BB, _HID + _EMB_DIM), lambda i: (i, 0)),
        out_shape=jax.ShapeDtypeStruct((_BATCH, _HID + _EMB_DIM), jnp.float32),
    )(feat, W, b.reshape(1, _HID), emb)
    return out
